# Initial kernel scaffold; baseline (speedup 1.0000x reference)
import jax
import jax.numpy as jnp
from jax.experimental import pallas as pl


def kernel(offset, mask, relpos_W, mask_W):
    raise NotImplementedError("write your pallas kernel here")


# speedup vs baseline: 1.6297x; 1.6297x over previous
import functools

import jax
import jax.numpy as jnp
from jax import lax
from jax.experimental import pallas as pl
from jax.experimental.pallas import tpu as pltpu
from jax.experimental.pallas import tpu_sc as plsc

_MAX_REL = 32
_NROWS = 2 * _MAX_REL + 1
_D = 128
_CROWS = 2 * _NROWS

_NC = 2
_NS = 16
_NW = _NC * _NS
_L = 16

_K = 256


@functools.partial(jax.jit, static_argnums=(4,))
def _run(off_flat, msk_flat, rel_flat, mw_flat, n_tokens):
    per_w = n_tokens // _NW
    n_chunks = per_w // _K
    mesh = plsc.VectorSubcoreMesh(
        core_axis_name="c", subcore_axis_name="s",
        num_cores=_NC, num_subcores=_NS,
    )

    @functools.partial(
        pl.kernel,
        out_type=jax.ShapeDtypeStruct((n_tokens * _D,), jnp.float32),
        mesh=mesh,
        compiler_params=pltpu.CompilerParams(needs_layout_passes=False),
        scratch_types=[
            pltpu.VMEM((_NROWS * _D,), jnp.float32),
            pltpu.VMEM((2 * _D,), jnp.float32),
            pltpu.VMEM((_CROWS * _D,), jnp.float32),
            pltpu.VMEM((_K,), jnp.int32),
            pltpu.VMEM((_K,), jnp.int32),
            pltpu.VMEM((_K * _D,), jnp.float32),
        ],
    )
    def k(off_h, msk_h, rel_h, mw_h, out_h, rel_v, mw_v, c_v, off_v, msk_v, stage_v):
        wid = lax.axis_index("s") * _NC + lax.axis_index("c")
        pltpu.sync_copy(rel_h, rel_v)
        pltpu.sync_copy(mw_h, mw_v)

        def build(i, _):
            d = i // (_D // _L)
            g = i % (_D // _L)
            r = rel_v[pl.ds(d * _D + g * _L, _L)]
            m0 = mw_v[pl.ds(g * _L, _L)]
            m1 = mw_v[pl.ds(_D + g * _L, _L)]
            c_v[pl.ds(d * 2 * _D + g * _L, _L)] = r + m0
            c_v[pl.ds(d * 2 * _D + _D + g * _L, _L)] = r + m1
            return 0

        lax.fori_loop(0, _NROWS * (_D // _L), build, 0)

        lane = lax.iota(jnp.int32, _L)

        def chunk(ch, _):
            base = wid * per_w + ch * _K
            pltpu.sync_copy(off_h.at[pl.ds(base, _K)], off_v)
            pltpu.sync_copy(msk_h.at[pl.ds(base, _K)], msk_v)

            def group(t, _):
                off = off_v[pl.ds(t * _L, _L)]
                m = msk_v[pl.ds(t * _L, _L)]
                d = jnp.clip(off + _MAX_REL, 0, 2 * _MAX_REL)
                src = d * (2 * _D) + m * _D
                dst = (t * _L + lane) * _D

                def col(c, _):
                    v = plsc.load_gather(c_v, [src + c])
                    plsc.store_scatter(stage_v, [dst + c], v)
                    return 0

                lax.fori_loop(0, _D, col, 0)
                return 0

            lax.fori_loop(0, _K // _L, group, 0)
            pltpu.sync_copy(stage_v, out_h.at[pl.ds(base * _D, _K * _D)])
            return 0

        lax.fori_loop(0, n_chunks, chunk, 0)

    return k(off_flat, msk_flat, rel_flat, mw_flat)


def kernel(offset, mask, relpos_W, mask_W):
    n = offset.size
    out = _run(
        offset.reshape(-1),
        mask.reshape(-1),
        relpos_W.reshape(-1),
        mask_W.reshape(-1),
        n,
    )
    return out.reshape(*offset.shape, _D)


# speedup vs baseline: 1.7338x; 1.0639x over previous
import functools

import jax
import jax.numpy as jnp
from jax import lax
from jax.experimental import pallas as pl
from jax.experimental.pallas import tpu as pltpu
from jax.experimental.pallas import tpu_sc as plsc

_MAX_REL = 32
_NROWS = 2 * _MAX_REL + 1
_D = 128
_CROWS = 2 * _NROWS

_NC = 2
_NS = 16
_NW = _NC * _NS
_L = 16

_K = 320


@functools.partial(jax.jit, static_argnums=(4,))
def _run(off_flat, msk_flat, rel_flat, mw_flat, n_tokens):
    per_w = n_tokens // _NW
    n_chunks = per_w // _K
    mesh = plsc.VectorSubcoreMesh(
        core_axis_name="c", subcore_axis_name="s",
        num_cores=_NC, num_subcores=_NS,
    )

    @functools.partial(
        pl.kernel,
        out_type=jax.ShapeDtypeStruct((n_tokens * _D,), jnp.float32),
        mesh=mesh,
        compiler_params=pltpu.CompilerParams(needs_layout_passes=False),
        scratch_types=[
            pltpu.VMEM((_NROWS * _D,), jnp.float32),
            pltpu.VMEM((2 * _D,), jnp.float32),
            pltpu.VMEM((_CROWS * _D,), jnp.float32),
            pltpu.VMEM((2 * _K,), jnp.int32),
            pltpu.VMEM((2 * _K,), jnp.int32),
            pltpu.VMEM((2 * _K * _D,), jnp.float32),
            pltpu.SemaphoreType.DMA,
            pltpu.SemaphoreType.DMA,
            pltpu.SemaphoreType.DMA,
            pltpu.SemaphoreType.DMA,
        ],
    )
    def k(off_h, msk_h, rel_h, mw_h, out_h,
          rel_v, mw_v, c_v, off_v, msk_v, stage_v,
          isem0, isem1, osem0, osem1):
        wid = lax.axis_index("s") * _NC + lax.axis_index("c")
        pltpu.sync_copy(rel_h, rel_v)
        pltpu.sync_copy(mw_h, mw_v)

        def build(i, _):
            d = i // (_D // _L)
            g = i % (_D // _L)
            r = rel_v[pl.ds(d * _D + g * _L, _L)]
            m0 = mw_v[pl.ds(g * _L, _L)]
            m1 = mw_v[pl.ds(_D + g * _L, _L)]
            c_v[pl.ds(d * 2 * _D + g * _L, _L)] = r + m0
            c_v[pl.ds(d * 2 * _D + _D + g * _L, _L)] = r + m1
            return 0

        lax.fori_loop(0, _NROWS * (_D // _L), build, 0)

        lane = lax.iota(jnp.int32, _L)
        isems = (isem0, isem1)
        osems = (osem0, osem1)
        tok0 = wid * per_w

        def idx_start(ch, b):
            base = tok0 + ch * _K
            pltpu.make_async_copy(
                off_h.at[pl.ds(base, _K)], off_v.at[pl.ds(b * _K, _K)], isems[b]).start()
            pltpu.make_async_copy(
                msk_h.at[pl.ds(base, _K)], msk_v.at[pl.ds(b * _K, _K)], isems[b]).start()

        def idx_wait(ch, b):
            base = tok0 + ch * _K
            pltpu.make_async_copy(
                off_h.at[pl.ds(base, _K)], off_v.at[pl.ds(b * _K, _K)], isems[b]).wait()
            pltpu.make_async_copy(
                msk_h.at[pl.ds(base, _K)], msk_v.at[pl.ds(b * _K, _K)], isems[b]).wait()

        def out_wait(b):
            pltpu.make_async_copy(
                stage_v.at[pl.ds(b * _K * _D, _K * _D)],
                out_h.at[pl.ds(0, _K * _D)], osems[b]).wait()

        idx_start(0, 0)
        idx_start(1, 1)

        def pair(p, _):
            for b in range(2):
                ch = p * 2 + b
                idx_wait(ch, b)

                @pl.when(ch >= 2)
                def _():
                    out_wait(b)

                def group(t, _):
                    off = off_v[pl.ds(b * _K + t * _L, _L)]
                    m = msk_v[pl.ds(b * _K + t * _L, _L)]
                    d = jnp.clip(off + _MAX_REL, 0, 2 * _MAX_REL)
                    src = d * (2 * _D) + m * _D
                    dst = (t * _L + lane) * _D + b * _K * _D
                    for c in range(_D):
                        v = plsc.load_gather(c_v, [src + c])
                        plsc.store_scatter(stage_v, [dst + c], v)
                    return 0

                lax.fori_loop(0, _K // _L, group, 0)

                base = tok0 + ch * _K
                pltpu.make_async_copy(
                    stage_v.at[pl.ds(b * _K * _D, _K * _D)],
                    out_h.at[pl.ds(base * _D, _K * _D)],
                    osems[b]).start()

                @pl.when(ch + 2 < n_chunks)
                def _():
                    idx_start(ch + 2, b)
            return 0

        lax.fori_loop(0, n_chunks // 2, pair, 0)
        out_wait(0)
        out_wait(1)

    return k(off_flat, msk_flat, rel_flat, mw_flat)


def kernel(offset, mask, relpos_W, mask_W):
    n = offset.size
    out = _run(
        offset.reshape(-1),
        mask.reshape(-1),
        relpos_W.reshape(-1),
        mask_W.reshape(-1),
        n,
    )
    return out.reshape(*offset.shape, _D)


# speedup vs baseline: 7.6387x; 4.4059x over previous
import functools

import jax
import jax.numpy as jnp
from jax import lax
from jax.experimental import pallas as pl
from jax.experimental.pallas import tpu as pltpu
from jax.experimental.pallas import tpu_sc as plsc

_MAX_REL = 32
_NROWS = 2 * _MAX_REL + 1
_D = 128
_CROWS = 2 * _NROWS

_NC = 2
_NS = 16
_NW = _NC * _NS
_L = 16

_K = 320


@functools.partial(jax.jit, static_argnums=(4,))
def _run(off_flat, msk_flat, rel_flat, mw_flat, n_tokens):
    per_w = n_tokens // _NW
    n_chunks = per_w // _K
    mesh = plsc.VectorSubcoreMesh(
        core_axis_name="c", subcore_axis_name="s",
        num_cores=_NC, num_subcores=_NS,
    )

    @functools.partial(
        pl.kernel,
        out_type=jax.ShapeDtypeStruct((n_tokens * _D,), jnp.float32),
        mesh=mesh,
        compiler_params=pltpu.CompilerParams(needs_layout_passes=False),
        scratch_types=[
            pltpu.VMEM((_NROWS * _D,), jnp.float32),
            pltpu.VMEM((2 * _D,), jnp.float32),
            pltpu.VMEM((_CROWS * _D,), jnp.float32),
            pltpu.VMEM((2 * _K,), jnp.int32),
            pltpu.VMEM((2 * _K,), jnp.int32),
            pltpu.VMEM((2 * _K * _D,), jnp.float32),
            pltpu.SemaphoreType.DMA,
            pltpu.SemaphoreType.DMA,
            pltpu.SemaphoreType.DMA,
            pltpu.SemaphoreType.DMA,
        ],
    )
    def k(off_h, msk_h, rel_h, mw_h, out_h,
          rel_v, mw_v, c_v, off_v, msk_v, stage_v,
          isem0, isem1, osem0, osem1):
        wid = lax.axis_index("s") * _NC + lax.axis_index("c")
        pltpu.sync_copy(rel_h, rel_v)
        pltpu.sync_copy(mw_h, mw_v)

        def build(i, _):
            d = i // (_D // _L)
            g = i % (_D // _L)
            r = rel_v[pl.ds(d * _D + g * _L, _L)]
            m0 = mw_v[pl.ds(g * _L, _L)]
            m1 = mw_v[pl.ds(_D + g * _L, _L)]
            c_v[pl.ds(d * 2 * _D + g * _L, _L)] = r + m0
            c_v[pl.ds(d * 2 * _D + _D + g * _L, _L)] = r + m1
            return 0

        lax.fori_loop(0, _NROWS * (_D // _L), build, 0)

        lane = lax.iota(jnp.int32, _L)
        isems = (isem0, isem1)
        osems = (osem0, osem1)
        tok0 = wid * per_w

        def idx_start(ch, b):
            base = tok0 + ch * _K
            pltpu.make_async_copy(
                off_h.at[pl.ds(base, _K)], off_v.at[pl.ds(b * _K, _K)], isems[b]).start()
            pltpu.make_async_copy(
                msk_h.at[pl.ds(base, _K)], msk_v.at[pl.ds(b * _K, _K)], isems[b]).start()

        def idx_wait(ch, b):
            base = tok0 + ch * _K
            pltpu.make_async_copy(
                off_h.at[pl.ds(base, _K)], off_v.at[pl.ds(b * _K, _K)], isems[b]).wait()
            pltpu.make_async_copy(
                msk_h.at[pl.ds(base, _K)], msk_v.at[pl.ds(b * _K, _K)], isems[b]).wait()

        def out_wait(b):
            pltpu.make_async_copy(
                stage_v.at[pl.ds(b * _K * _D, _K * _D)],
                out_h.at[pl.ds(0, _K * _D)], osems[b]).wait()

        idx_start(0, 0)
        idx_start(1, 1)

        def pair(p, _):
            for b in range(2):
                ch = p * 2 + b
                idx_wait(ch, b)

                @pl.when(ch >= 2)
                def _():
                    out_wait(b)

                def group(t, _):
                    off = off_v[pl.ds(b * _K + t * _L, _L)]
                    m = msk_v[pl.ds(b * _K + t * _L, _L)]
                    d = jnp.clip(off + _MAX_REL, 0, 2 * _MAX_REL)
                    src = d * (2 * _D) + m * _D
                    dst = (t * _L + lane) * _D + b * _K * _D
                    for c in range(_D):
                        col = (lane + c) & (_D - 1)
                        v = plsc.load_gather(c_v, [src + col])
                        plsc.store_scatter(stage_v, [dst + col], v)
                    return 0

                lax.fori_loop(0, _K // _L, group, 0)

                base = tok0 + ch * _K
                pltpu.make_async_copy(
                    stage_v.at[pl.ds(b * _K * _D, _K * _D)],
                    out_h.at[pl.ds(base * _D, _K * _D)],
                    osems[b]).start()

                @pl.when(ch + 2 < n_chunks)
                def _():
                    idx_start(ch + 2, b)
            return 0

        lax.fori_loop(0, n_chunks // 2, pair, 0)
        out_wait(0)
        out_wait(1)

    return k(off_flat, msk_flat, rel_flat, mw_flat)


def kernel(offset, mask, relpos_W, mask_W):
    n = offset.size
    out = _run(
        offset.reshape(-1),
        mask.reshape(-1),
        relpos_W.reshape(-1),
        mask_W.reshape(-1),
        n,
    )
    return out.reshape(*offset.shape, _D)


# speedup vs baseline: 29.7621x; 3.8962x over previous
import functools

import jax
import jax.numpy as jnp
from jax import lax
from jax.experimental import pallas as pl
from jax.experimental.pallas import tpu as pltpu
from jax.experimental.pallas import tpu_sc as plsc

_MAX_REL = 32
_NROWS = 2 * _MAX_REL + 1
_D = 128
_CROWS = 2 * _NROWS

_NC = 2
_NS = 16
_NW = _NC * _NS
_L = 16

_K = 320


@functools.partial(jax.jit, static_argnums=(4,))
def _run(off_flat, msk_flat, rel_flat, mw_flat, n_tokens):
    per_w = n_tokens // _NW
    n_chunks = per_w // _K
    mesh = plsc.VectorSubcoreMesh(
        core_axis_name="c", subcore_axis_name="s",
        num_cores=_NC, num_subcores=_NS,
    )

    @functools.partial(
        pl.kernel,
        out_type=jax.ShapeDtypeStruct((n_tokens * _D,), jnp.float32),
        mesh=mesh,
        compiler_params=pltpu.CompilerParams(needs_layout_passes=False),
        scratch_types=[
            pltpu.VMEM((_NROWS * _D,), jnp.float32),
            pltpu.VMEM((2 * _D,), jnp.float32),
            pltpu.VMEM((_CROWS * _D,), jnp.float32),
            pltpu.VMEM((2 * _K,), jnp.int32),
            pltpu.VMEM((2 * _K,), jnp.int32),
            pltpu.VMEM((2 * _K * _D,), jnp.float32),
            pltpu.SemaphoreType.DMA,
            pltpu.SemaphoreType.DMA,
            pltpu.SemaphoreType.DMA,
            pltpu.SemaphoreType.DMA,
        ],
    )
    def k(off_h, msk_h, rel_h, mw_h, out_h,
          rel_v, mw_v, c_v, off_v, msk_v, stage_v,
          isem0, isem1, osem0, osem1):
        wid = lax.axis_index("s") * _NC + lax.axis_index("c")
        pltpu.sync_copy(rel_h, rel_v)
        pltpu.sync_copy(mw_h, mw_v)

        def build(i, _):
            d = i // (_D // _L)
            g = i % (_D // _L)
            r = rel_v[pl.ds(d * _D + g * _L, _L)]
            m0 = mw_v[pl.ds(g * _L, _L)]
            m1 = mw_v[pl.ds(_D + g * _L, _L)]
            c_v[pl.ds(d * 2 * _D + g * _L, _L)] = r + m0
            c_v[pl.ds(d * 2 * _D + _D + g * _L, _L)] = r + m1
            return 0

        lax.fori_loop(0, _NROWS * (_D // _L), build, 0)

        lane = lax.iota(jnp.int32, _L)
        isems = (isem0, isem1)
        osems = (osem0, osem1)
        tok0 = wid * per_w

        def idx_start(ch, b):
            base = tok0 + ch * _K
            pltpu.make_async_copy(
                off_h.at[pl.ds(base, _K)], off_v.at[pl.ds(b * _K, _K)], isems[b]).start()
            pltpu.make_async_copy(
                msk_h.at[pl.ds(base, _K)], msk_v.at[pl.ds(b * _K, _K)], isems[b]).start()

        def idx_wait(ch, b):
            base = tok0 + ch * _K
            pltpu.make_async_copy(
                off_h.at[pl.ds(base, _K)], off_v.at[pl.ds(b * _K, _K)], isems[b]).wait()
            pltpu.make_async_copy(
                msk_h.at[pl.ds(base, _K)], msk_v.at[pl.ds(b * _K, _K)], isems[b]).wait()

        def out_wait(b):
            pltpu.make_async_copy(
                stage_v.at[pl.ds(b * _K * _D, _K * _D)],
                out_h.at[pl.ds(0, _K * _D)], osems[b]).wait()

        idx_start(0, 0)
        idx_start(1, 1)

        def pair(p, _):
            for b in range(2):
                ch = p * 2 + b
                idx_wait(ch, b)

                @pl.when(ch >= 2)
                def _():
                    out_wait(b)

                def group(t, _):
                    off = off_v[pl.ds(b * _K + t * _L, _L)]
                    m = msk_v[pl.ds(b * _K + t * _L, _L)]
                    d = jnp.clip(off + _MAX_REL, 0, 2 * _MAX_REL)
                    src = d * (2 * _D) + m * _D
                    dst = (t * _L + lane) * _D + b * _K * _D
                    @plsc.parallel_loop(0, _D, 1, unroll=8)
                    def _(c):
                        col = (lane + c) & (_D - 1)
                        v = plsc.load_gather(c_v, [src + col])
                        plsc.store_scatter(stage_v, [dst + col], v)
                    return 0

                lax.fori_loop(0, _K // _L, group, 0)

                base = tok0 + ch * _K
                pltpu.make_async_copy(
                    stage_v.at[pl.ds(b * _K * _D, _K * _D)],
                    out_h.at[pl.ds(base * _D, _K * _D)],
                    osems[b]).start()

                @pl.when(ch + 2 < n_chunks)
                def _():
                    idx_start(ch + 2, b)
            return 0

        lax.fori_loop(0, n_chunks // 2, pair, 0)
        out_wait(0)
        out_wait(1)

    return k(off_flat, msk_flat, rel_flat, mw_flat)


def kernel(offset, mask, relpos_W, mask_W):
    n = offset.size
    out = _run(
        offset.reshape(-1),
        mask.reshape(-1),
        relpos_W.reshape(-1),
        mask_W.reshape(-1),
        n,
    )
    return out.reshape(*offset.shape, _D)


# speedup vs baseline: 32.2084x; 1.0822x over previous
import functools

import jax
import jax.numpy as jnp
from jax import lax
from jax.experimental import pallas as pl
from jax.experimental.pallas import tpu as pltpu
from jax.experimental.pallas import tpu_sc as plsc

_MAX_REL = 32
_NROWS = 2 * _MAX_REL + 1
_D = 128
_CROWS = 2 * _NROWS

_NC = 2
_NS = 16
_NW = _NC * _NS
_L = 16

_K = 320


@functools.partial(jax.jit, static_argnums=(4,))
def _run(off_flat, msk_flat, rel_flat, mw_flat, n_tokens):
    per_w = n_tokens // _NW
    n_chunks = per_w // _K
    mesh = plsc.VectorSubcoreMesh(
        core_axis_name="c", subcore_axis_name="s",
        num_cores=_NC, num_subcores=_NS,
    )

    @functools.partial(
        pl.kernel,
        out_type=jax.ShapeDtypeStruct((n_tokens * _D,), jnp.float32),
        mesh=mesh,
        compiler_params=pltpu.CompilerParams(needs_layout_passes=False),
        scratch_types=[
            pltpu.VMEM((_NROWS * _D,), jnp.float32),
            pltpu.VMEM((2 * _D,), jnp.float32),
            pltpu.VMEM((_CROWS * _D,), jnp.float32),
            pltpu.VMEM((2 * _K,), jnp.int32),
            pltpu.VMEM((2 * _K,), jnp.int32),
            pltpu.VMEM((2 * _K * _D,), jnp.float32),
            pltpu.SemaphoreType.DMA,
            pltpu.SemaphoreType.DMA,
            pltpu.SemaphoreType.DMA,
            pltpu.SemaphoreType.DMA,
        ],
    )
    def k(off_h, msk_h, rel_h, mw_h, out_h,
          rel_v, mw_v, c_v, off_v, msk_v, stage_v,
          isem0, isem1, osem0, osem1):
        wid = lax.axis_index("s") * _NC + lax.axis_index("c")
        pltpu.sync_copy(rel_h, rel_v)
        pltpu.sync_copy(mw_h, mw_v)

        def build(i, _):
            d = i // (_D // _L)
            g = i % (_D // _L)
            r = rel_v[pl.ds(d * _D + g * _L, _L)]
            m0 = mw_v[pl.ds(g * _L, _L)]
            m1 = mw_v[pl.ds(_D + g * _L, _L)]
            c_v[pl.ds(d * 2 * _D + g * _L, _L)] = r + m0
            c_v[pl.ds(d * 2 * _D + _D + g * _L, _L)] = r + m1
            return 0

        lax.fori_loop(0, _NROWS * (_D // _L), build, 0)

        lane = lax.iota(jnp.int32, _L)
        isems = (isem0, isem1)
        osems = (osem0, osem1)
        tok0 = wid * per_w

        def idx_start(ch, b):
            base = tok0 + ch * _K
            pltpu.make_async_copy(
                off_h.at[pl.ds(base, _K)], off_v.at[pl.ds(b * _K, _K)], isems[b]).start()
            pltpu.make_async_copy(
                msk_h.at[pl.ds(base, _K)], msk_v.at[pl.ds(b * _K, _K)], isems[b]).start()

        def idx_wait(ch, b):
            base = tok0 + ch * _K
            pltpu.make_async_copy(
                off_h.at[pl.ds(base, _K)], off_v.at[pl.ds(b * _K, _K)], isems[b]).wait()
            pltpu.make_async_copy(
                msk_h.at[pl.ds(base, _K)], msk_v.at[pl.ds(b * _K, _K)], isems[b]).wait()

        def out_wait(b):
            pltpu.make_async_copy(
                stage_v.at[pl.ds(b * _K * _D, _K * _D)],
                out_h.at[pl.ds(0, _K * _D)], osems[b]).wait()

        idx_start(0, 0)
        idx_start(1, 1)

        def pair(p, _):
            for b in range(2):
                ch = p * 2 + b
                idx_wait(ch, b)

                @pl.when(ch >= 2)
                def _():
                    out_wait(b)

                def group(t, _):
                    off = off_v[pl.ds(b * _K + t * _L, _L)]
                    m = msk_v[pl.ds(b * _K + t * _L, _L)]
                    d = jnp.clip(off + _MAX_REL, 0, 2 * _MAX_REL)
                    src = d * (2 * _D) + m * _D
                    dst = (t * _L + lane) * _D + b * _K * _D
                    @plsc.parallel_loop(0, _D, 1, unroll=16)
                    def _(c):
                        col = (lane + c) & (_D - 1)
                        v = plsc.load_gather(c_v, [src + col])
                        plsc.store_scatter(stage_v, [dst + col], v)
                    return 0

                lax.fori_loop(0, _K // _L, group, 0)

                base = tok0 + ch * _K
                pltpu.make_async_copy(
                    stage_v.at[pl.ds(b * _K * _D, _K * _D)],
                    out_h.at[pl.ds(base * _D, _K * _D)],
                    osems[b]).start()

                @pl.when(ch + 2 < n_chunks)
                def _():
                    idx_start(ch + 2, b)
            return 0

        lax.fori_loop(0, n_chunks // 2, pair, 0)
        out_wait(0)
        out_wait(1)

    return k(off_flat, msk_flat, rel_flat, mw_flat)


def kernel(offset, mask, relpos_W, mask_W):
    n = offset.size
    out = _run(
        offset.reshape(-1),
        mask.reshape(-1),
        relpos_W.reshape(-1),
        mask_W.reshape(-1),
        n,
    )
    return out.reshape(*offset.shape, _D)


# speedup vs baseline: 32.2616x; 1.0017x over previous
import functools

import jax
import jax.numpy as jnp
from jax import lax
from jax.experimental import pallas as pl
from jax.experimental.pallas import tpu as pltpu
from jax.experimental.pallas import tpu_sc as plsc

_MAX_REL = 32
_NROWS = 2 * _MAX_REL + 1
_D = 128
_CROWS = 2 * _NROWS

_NC = 2
_NS = 16
_NW = _NC * _NS
_L = 16

_K = 320


@functools.partial(jax.jit, static_argnums=(4,))
def _run(off_flat, msk_flat, rel_flat, mw_flat, n_tokens):
    per_w = n_tokens // _NW
    n_chunks = per_w // _K
    mesh = plsc.VectorSubcoreMesh(
        core_axis_name="c", subcore_axis_name="s",
        num_cores=_NC, num_subcores=_NS,
    )

    @functools.partial(
        pl.kernel,
        out_type=jax.ShapeDtypeStruct((n_tokens * _D,), jnp.float32),
        mesh=mesh,
        compiler_params=pltpu.CompilerParams(needs_layout_passes=False),
        scratch_types=[
            pltpu.VMEM((_NROWS * _D,), jnp.float32),
            pltpu.VMEM((2 * _D,), jnp.float32),
            pltpu.VMEM((_CROWS * _D,), jnp.float32),
            pltpu.VMEM((2 * _K,), jnp.int32),
            pltpu.VMEM((2 * _K,), jnp.int32),
            pltpu.VMEM((2 * _K * _D,), jnp.float32),
            pltpu.SemaphoreType.DMA,
            pltpu.SemaphoreType.DMA,
            pltpu.SemaphoreType.DMA,
            pltpu.SemaphoreType.DMA,
        ],
    )
    def k(off_h, msk_h, rel_h, mw_h, out_h,
          rel_v, mw_v, c_v, off_v, msk_v, stage_v,
          isem0, isem1, osem0, osem1):
        wid = lax.axis_index("s") * _NC + lax.axis_index("c")
        pltpu.sync_copy(rel_h, rel_v)
        pltpu.sync_copy(mw_h, mw_v)

        def build(i, _):
            d = i // (_D // _L)
            g = i % (_D // _L)
            r = rel_v[pl.ds(d * _D + g * _L, _L)]
            m0 = mw_v[pl.ds(g * _L, _L)]
            m1 = mw_v[pl.ds(_D + g * _L, _L)]
            c_v[pl.ds(d * 2 * _D + g * _L, _L)] = r + m0
            c_v[pl.ds(d * 2 * _D + _D + g * _L, _L)] = r + m1
            return 0

        lax.fori_loop(0, _NROWS * (_D // _L), build, 0)

        lane = lax.iota(jnp.int32, _L)
        isems = (isem0, isem1)
        osems = (osem0, osem1)
        tok0 = wid * per_w

        def idx_start(ch, b):
            base = tok0 + ch * _K
            pltpu.make_async_copy(
                off_h.at[pl.ds(base, _K)], off_v.at[pl.ds(b * _K, _K)], isems[b]).start()
            pltpu.make_async_copy(
                msk_h.at[pl.ds(base, _K)], msk_v.at[pl.ds(b * _K, _K)], isems[b]).start()

        def idx_wait(ch, b):
            base = tok0 + ch * _K
            pltpu.make_async_copy(
                off_h.at[pl.ds(base, _K)], off_v.at[pl.ds(b * _K, _K)], isems[b]).wait()
            pltpu.make_async_copy(
                msk_h.at[pl.ds(base, _K)], msk_v.at[pl.ds(b * _K, _K)], isems[b]).wait()

        def out_wait(b):
            pltpu.make_async_copy(
                stage_v.at[pl.ds(b * _K * _D, _K * _D)],
                out_h.at[pl.ds(0, _K * _D)], osems[b]).wait()

        idx_start(0, 0)
        idx_start(1, 1)

        def pair(p, _):
            for b in range(2):
                ch = p * 2 + b
                idx_wait(ch, b)

                @pl.when(ch >= 2)
                def _():
                    out_wait(b)

                @plsc.parallel_loop(0, _K // _L, 1)
                def group(t):
                    off = off_v[pl.ds(b * _K + t * _L, _L)]
                    m = msk_v[pl.ds(b * _K + t * _L, _L)]
                    d = jnp.clip(off + _MAX_REL, 0, 2 * _MAX_REL)
                    src = d * (2 * _D) + m * _D
                    dst = (t * _L + lane) * _D + b * _K * _D
                    @plsc.parallel_loop(0, _D, 1, unroll=16)
                    def _(c):
                        col = (lane + c) & (_D - 1)
                        v = plsc.load_gather(c_v, [src + col])
                        plsc.store_scatter(stage_v, [dst + col], v)

                base = tok0 + ch * _K
                pltpu.make_async_copy(
                    stage_v.at[pl.ds(b * _K * _D, _K * _D)],
                    out_h.at[pl.ds(base * _D, _K * _D)],
                    osems[b]).start()

                @pl.when(ch + 2 < n_chunks)
                def _():
                    idx_start(ch + 2, b)
            return 0

        lax.fori_loop(0, n_chunks // 2, pair, 0)
        out_wait(0)
        out_wait(1)

    return k(off_flat, msk_flat, rel_flat, mw_flat)


def kernel(offset, mask, relpos_W, mask_W):
    n = offset.size
    out = _run(
        offset.reshape(-1),
        mask.reshape(-1),
        relpos_W.reshape(-1),
        mask_W.reshape(-1),
        n,
    )
    return out.reshape(*offset.shape, _D)


# speedup vs baseline: 36.5441x; 1.1327x over previous
import functools

import jax
import jax.numpy as jnp
from jax import lax
from jax.experimental import pallas as pl
from jax.experimental.pallas import tpu as pltpu
from jax.experimental.pallas import tpu_sc as plsc

_MAX_REL = 32
_NROWS = 2 * _MAX_REL + 1
_D = 128
_CROWS = 2 * _NROWS

_NC = 2
_NS = 16
_NW = _NC * _NS
_L = 16

_K = 320


@functools.partial(jax.jit, static_argnums=(4,))
def _run(off_flat, msk_flat, rel_flat, mw_flat, n_tokens):
    per_w = n_tokens // _NW
    n_chunks = per_w // _K
    mesh = plsc.VectorSubcoreMesh(
        core_axis_name="c", subcore_axis_name="s",
        num_cores=_NC, num_subcores=_NS,
    )

    @functools.partial(
        pl.kernel,
        out_type=jax.ShapeDtypeStruct((n_tokens * _D,), jnp.float32),
        mesh=mesh,
        compiler_params=pltpu.CompilerParams(needs_layout_passes=False),
        scratch_types=[
            pltpu.VMEM((_NROWS * _D,), jnp.float32),
            pltpu.VMEM((2 * _D,), jnp.float32),
            pltpu.VMEM((_CROWS * _D,), jnp.float32),
            pltpu.VMEM((2 * _K,), jnp.int32),
            pltpu.VMEM((2 * _K,), jnp.int32),
            pltpu.VMEM((2 * _K * _D,), jnp.float32),
            pltpu.SemaphoreType.DMA,
            pltpu.SemaphoreType.DMA,
            pltpu.SemaphoreType.DMA,
            pltpu.SemaphoreType.DMA,
        ],
    )
    def k(off_h, msk_h, rel_h, mw_h, out_h,
          rel_v, mw_v, c_v, off_v, msk_v, stage_v,
          isem0, isem1, osem0, osem1):
        wid = lax.axis_index("s") * _NC + lax.axis_index("c")
        pltpu.sync_copy(rel_h, rel_v)
        pltpu.sync_copy(mw_h, mw_v)

        def build(i, _):
            d = i // (_D // _L)
            g = i % (_D // _L)
            r = rel_v[pl.ds(d * _D + g * _L, _L)]
            m0 = mw_v[pl.ds(g * _L, _L)]
            m1 = mw_v[pl.ds(_D + g * _L, _L)]
            c_v[pl.ds(d * 2 * _D + g * _L, _L)] = r + m0
            c_v[pl.ds(d * 2 * _D + _D + g * _L, _L)] = r + m1
            return 0

        lax.fori_loop(0, _NROWS * (_D // _L), build, 0)

        lane = lax.iota(jnp.int32, _L)
        isems = (isem0, isem1)
        osems = (osem0, osem1)
        tok0 = wid * per_w

        def idx_start(ch, b):
            base = tok0 + ch * _K
            pltpu.make_async_copy(
                off_h.at[pl.ds(base, _K)], off_v.at[pl.ds(b * _K, _K)], isems[b]).start()
            pltpu.make_async_copy(
                msk_h.at[pl.ds(base, _K)], msk_v.at[pl.ds(b * _K, _K)], isems[b]).start()

        def idx_wait(ch, b):
            base = tok0 + ch * _K
            pltpu.make_async_copy(
                off_h.at[pl.ds(base, _K)], off_v.at[pl.ds(b * _K, _K)], isems[b]).wait()
            pltpu.make_async_copy(
                msk_h.at[pl.ds(base, _K)], msk_v.at[pl.ds(b * _K, _K)], isems[b]).wait()

        def out_wait(b):
            pltpu.make_async_copy(
                stage_v.at[pl.ds(b * _K * _D, _K * _D)],
                out_h.at[pl.ds(0, _K * _D)], osems[b]).wait()

        idx_start(0, 0)
        idx_start(1, 1)

        def pair(p, _):
            for b in range(2):
                ch = p * 2 + b
                idx_wait(ch, b)

                @pl.when(ch >= 2)
                def _():
                    out_wait(b)

                @plsc.parallel_loop(0, _K // _L, 1)
                def group(t):
                    off = off_v[pl.ds(b * _K + t * _L, _L)]
                    m = msk_v[pl.ds(b * _K + t * _L, _L)]
                    d = jnp.clip(off + _MAX_REL, 0, 2 * _MAX_REL)
                    src = d * (2 * _D) + m * _D
                    dst = (t * _L + lane) * _D + b * _K * _D
                    srcl = src + lane
                    dstl = dst + lane
                    @plsc.parallel_loop(0, 112, 1, unroll=16)
                    def _(c):
                        v = plsc.load_gather(c_v, [srcl + c])
                        plsc.store_scatter(stage_v, [dstl + c], v)
                    @plsc.parallel_loop(112, _D, 1, unroll=16)
                    def _(c):
                        col = (lane + c) & (_D - 1)
                        v = plsc.load_gather(c_v, [src + col])
                        plsc.store_scatter(stage_v, [dst + col], v)

                base = tok0 + ch * _K
                pltpu.make_async_copy(
                    stage_v.at[pl.ds(b * _K * _D, _K * _D)],
                    out_h.at[pl.ds(base * _D, _K * _D)],
                    osems[b]).start()

                @pl.when(ch + 2 < n_chunks)
                def _():
                    idx_start(ch + 2, b)
            return 0

        lax.fori_loop(0, n_chunks // 2, pair, 0)
        out_wait(0)
        out_wait(1)

    return k(off_flat, msk_flat, rel_flat, mw_flat)


def kernel(offset, mask, relpos_W, mask_W):
    n = offset.size
    out = _run(
        offset.reshape(-1),
        mask.reshape(-1),
        relpos_W.reshape(-1),
        mask_W.reshape(-1),
        n,
    )
    return out.reshape(*offset.shape, _D)


# speedup vs baseline: 36.7404x; 1.0054x over previous
import functools

import jax
import jax.numpy as jnp
from jax import lax
from jax.experimental import pallas as pl
from jax.experimental.pallas import tpu as pltpu
from jax.experimental.pallas import tpu_sc as plsc

_MAX_REL = 32
_NROWS = 2 * _MAX_REL + 1
_D = 128
_CROWS = 2 * _NROWS

_NC = 2
_NS = 16
_NW = _NC * _NS
_L = 16

_K = 400


@functools.partial(jax.jit, static_argnums=(4,))
def _run(off_flat, msk_flat, rel_flat, mw_flat, n_tokens):
    per_w = n_tokens // _NW
    n_chunks = per_w // _K
    mesh = plsc.VectorSubcoreMesh(
        core_axis_name="c", subcore_axis_name="s",
        num_cores=_NC, num_subcores=_NS,
    )

    @functools.partial(
        pl.kernel,
        out_type=jax.ShapeDtypeStruct((n_tokens * _D,), jnp.float32),
        mesh=mesh,
        compiler_params=pltpu.CompilerParams(needs_layout_passes=False),
        scratch_types=[
            pltpu.VMEM((_NROWS * _D,), jnp.float32),
            pltpu.VMEM((2 * _D,), jnp.float32),
            pltpu.VMEM((_CROWS * _D,), jnp.float32),
            pltpu.VMEM((2 * _K,), jnp.int32),
            pltpu.VMEM((2 * _K,), jnp.int32),
            pltpu.VMEM((2 * _K * _D,), jnp.float32),
            pltpu.SemaphoreType.DMA,
            pltpu.SemaphoreType.DMA,
            pltpu.SemaphoreType.DMA,
            pltpu.SemaphoreType.DMA,
        ],
    )
    def k(off_h, msk_h, rel_h, mw_h, out_h,
          rel_v, mw_v, c_v, off_v, msk_v, stage_v,
          isem0, isem1, osem0, osem1):
        wid = lax.axis_index("s") * _NC + lax.axis_index("c")
        pltpu.sync_copy(rel_h, rel_v)
        pltpu.sync_copy(mw_h, mw_v)

        def build(i, _):
            d = i // (_D // _L)
            g = i % (_D // _L)
            r = rel_v[pl.ds(d * _D + g * _L, _L)]
            m0 = mw_v[pl.ds(g * _L, _L)]
            m1 = mw_v[pl.ds(_D + g * _L, _L)]
            c_v[pl.ds(d * 2 * _D + g * _L, _L)] = r + m0
            c_v[pl.ds(d * 2 * _D + _D + g * _L, _L)] = r + m1
            return 0

        lax.fori_loop(0, _NROWS * (_D // _L), build, 0)

        lane = lax.iota(jnp.int32, _L)
        isems = (isem0, isem1)
        osems = (osem0, osem1)
        tok0 = wid * per_w

        def idx_start(ch, b):
            base = tok0 + ch * _K
            pltpu.make_async_copy(
                off_h.at[pl.ds(base, _K)], off_v.at[pl.ds(b * _K, _K)], isems[b]).start()
            pltpu.make_async_copy(
                msk_h.at[pl.ds(base, _K)], msk_v.at[pl.ds(b * _K, _K)], isems[b]).start()

        def idx_wait(ch, b):
            base = tok0 + ch * _K
            pltpu.make_async_copy(
                off_h.at[pl.ds(base, _K)], off_v.at[pl.ds(b * _K, _K)], isems[b]).wait()
            pltpu.make_async_copy(
                msk_h.at[pl.ds(base, _K)], msk_v.at[pl.ds(b * _K, _K)], isems[b]).wait()

        def out_wait(b):
            pltpu.make_async_copy(
                stage_v.at[pl.ds(b * _K * _D, _K * _D)],
                out_h.at[pl.ds(0, _K * _D)], osems[b]).wait()

        idx_start(0, 0)
        idx_start(1, 1)

        def pair(p, _):
            for b in range(2):
                ch = p * 2 + b
                idx_wait(ch, b)

                @pl.when(ch >= 2)
                def _():
                    out_wait(b)

                @plsc.parallel_loop(0, _K // _L, 1)
                def group(t):
                    off = off_v[pl.ds(b * _K + t * _L, _L)]
                    m = msk_v[pl.ds(b * _K + t * _L, _L)]
                    d = jnp.clip(off + _MAX_REL, 0, 2 * _MAX_REL)
                    src = d * (2 * _D) + m * _D
                    dst = (t * _L + lane) * _D + b * _K * _D
                    srcl = src + lane
                    dstl = dst + lane
                    @plsc.parallel_loop(0, 112, 1, unroll=16)
                    def _(c):
                        v = plsc.load_gather(c_v, [srcl + c])
                        plsc.store_scatter(stage_v, [dstl + c], v)
                    @plsc.parallel_loop(112, _D, 1, unroll=16)
                    def _(c):
                        col = (lane + c) & (_D - 1)
                        v = plsc.load_gather(c_v, [src + col])
                        plsc.store_scatter(stage_v, [dst + col], v)

                base = tok0 + ch * _K
                pltpu.make_async_copy(
                    stage_v.at[pl.ds(b * _K * _D, _K * _D)],
                    out_h.at[pl.ds(base * _D, _K * _D)],
                    osems[b]).start()

                @pl.when(ch + 2 < n_chunks)
                def _():
                    idx_start(ch + 2, b)
            return 0

        lax.fori_loop(0, n_chunks // 2, pair, 0)
        out_wait(0)
        out_wait(1)

    return k(off_flat, msk_flat, rel_flat, mw_flat)


def kernel(offset, mask, relpos_W, mask_W):
    n = offset.size
    out = _run(
        offset.reshape(-1),
        mask.reshape(-1),
        relpos_W.reshape(-1),
        mask_W.reshape(-1),
        n,
    )
    return out.reshape(*offset.shape, _D)
